# baseline (speedup 1.0000x reference)
import functools

import jax
import jax.numpy as jnp
from jax import lax
from jax.experimental import pallas as pl
from jax.experimental.pallas import tpu as pltpu
from jax.experimental.pallas import tpu_sc as plsc

N_TOK = 32768
D_MODEL = 768
N_EXPERTS = 64
TOP_K = 8
BT = 4096

NC = 2
NS = 16
L = 16
NW = NC * NS
TOK_PER_W = N_TOK // NW
CHUNK = 256
GROUPS = CHUNK // L


def _tc_body(x_ref, w_ref, b_ref, scores_ref, scores_t_ref):
    logits = jnp.dot(x_ref[...], w_ref[...], preferred_element_type=jnp.float32)
    scores = jax.nn.sigmoid(logits + b_ref[...])
    scores_ref[...] = scores
    scores_t_ref[...] = scores.T


def _tc_scores(x, W, b):
    return pl.pallas_call(
        _tc_body,
        grid=(N_TOK // BT,),
        in_specs=[
            pl.BlockSpec((BT, D_MODEL), lambda t: (t, 0)),
            pl.BlockSpec((D_MODEL, N_EXPERTS), lambda t: (0, 0)),
            pl.BlockSpec((1, N_EXPERTS), lambda t: (0, 0)),
        ],
        out_specs=[
            pl.BlockSpec((BT, N_EXPERTS), lambda t: (t, 0)),
            pl.BlockSpec((N_EXPERTS, BT), lambda t: (0, t)),
        ],
        out_shape=[
            jax.ShapeDtypeStruct((N_TOK, N_EXPERTS), jnp.float32),
            jax.ShapeDtypeStruct((N_EXPERTS, N_TOK), jnp.float32),
        ],
    )(x, W, b.reshape(1, N_EXPERTS))


@functools.partial(
    pl.kernel,
    out_type=[
        jax.ShapeDtypeStruct((TOP_K, N_TOK), jnp.float32),
        jax.ShapeDtypeStruct((TOP_K, N_TOK), jnp.int32),
    ],
    mesh=plsc.VectorSubcoreMesh(core_axis_name="c", subcore_axis_name="s"),
    scratch_types=[
        pltpu.VMEM((N_EXPERTS, CHUNK), jnp.float32),
        pltpu.VMEM((TOP_K, CHUNK), jnp.float32),
        pltpu.VMEM((TOP_K, CHUNK), jnp.int32),
    ],
)
def _sc_topk(scores_t_hbm, g_hbm, i_hbm, sc_v, g_v, i_v):
    wid = lax.axis_index("s") * NC + lax.axis_index("c")
    base_tok = wid * TOK_PER_W

    def chunk_body(ci, carry):
        tok0 = base_tok + ci * CHUNK
        pltpu.sync_copy(scores_t_hbm.at[:, pl.ds(tok0, CHUNK)], sc_v)

        def group_body(gi, c2):
            tbase = gi * L
            v = [jnp.full((L,), -2.0, jnp.float32) for _ in range(TOP_K)]
            ix = [jnp.zeros((L,), jnp.int32) for _ in range(TOP_K)]
            for e in range(N_EXPERTS):
                s = sc_v[e, pl.ds(tbase, L)]
                ei = jnp.full((L,), e, jnp.int32)
                for j in range(TOP_K):
                    m = s > v[j]
                    nv = jnp.where(m, s, v[j])
                    ns = jnp.where(m, v[j], s)
                    ni = jnp.where(m, ei, ix[j])
                    nei = jnp.where(m, ix[j], ei)
                    v[j] = nv
                    ix[j] = ni
                    s = ns
                    ei = nei
            total = v[0]
            for j in range(1, TOP_K):
                total = total + v[j]
            inv = 1.0 / total
            for j in range(TOP_K):
                g_v[j, pl.ds(tbase, L)] = v[j] * inv
                i_v[j, pl.ds(tbase, L)] = ix[j]
            return c2

        lax.fori_loop(0, GROUPS, group_body, 0)
        pltpu.sync_copy(g_v, g_hbm.at[:, pl.ds(tok0, CHUNK)])
        pltpu.sync_copy(i_v, i_hbm.at[:, pl.ds(tok0, CHUNK)])
        return carry

    lax.fori_loop(0, TOK_PER_W // CHUNK, chunk_body, 0)


def kernel(x, W, b):
    scores, scores_t = _tc_scores(x, W, b)
    g_t, i_t = _sc_topk(scores_t)
    return (g_t.T, i_t.T, scores)
